# baseline (speedup 1.0000x reference)
import functools

import jax
import jax.numpy as jnp
from jax import lax
from jax.experimental import pallas as pl
from jax.experimental.pallas import tpu as pltpu
from jax.experimental.pallas import tpu_sc as plsc

_NUM_WORKERS = 32
_CHUNK = 512
_NBUF = 2

_T = 200
_B = 16384
_D = 32


@functools.partial(jax.jit, static_argnums=())
def _gather_tiled(idx_flat, table):
    n_units = _T * (_B // _CHUNK)
    per_w = n_units // _NUM_WORKERS
    blocks_per_t = _B // _CHUNK
    n_out = _T * _D * _B
    mesh = plsc.VectorSubcoreMesh(core_axis_name="c", subcore_axis_name="s")

    scratch = (
        [pltpu.VMEM((_CHUNK,), jnp.int32) for _ in range(_NBUF)]
        + [pltpu.VMEM((_CHUNK, _D), jnp.float32) for _ in range(_NBUF)]
        + [pltpu.VMEM((4, 4096), jnp.float32) for _ in range(_NBUF)]
        + [pltpu.SemaphoreType.DMA for _ in range(2 * _NBUF)]
    )

    @functools.partial(
        pl.kernel,
        out_type=jax.ShapeDtypeStruct((n_out,), jnp.float32),
        mesh=mesh,
        scratch_types=scratch,
        compiler_params=pltpu.CompilerParams(
            use_tc_tiling_on_sc=False, needs_layout_passes=False
        ),
    )
    def gather_kernel(idx_hbm, table_hbm, out_hbm, *bufs):
        idx_v = bufs[:_NBUF]
        rows_v = bufs[_NBUF : 2 * _NBUF]
        outb = bufs[2 * _NBUF : 3 * _NBUF]
        gsem = bufs[3 * _NBUF : 4 * _NBUF]
        wsem = bufs[4 * _NBUF :]

        wid = lax.axis_index("s") * 2 + lax.axis_index("c")
        u0 = wid * per_w
        iota16 = lax.iota(jnp.int32, 16)
        iota_src = iota16 * _D

        def unit_coords(u):
            t = u // blocks_per_t
            g = u % blocks_per_t
            return t, g

        def stage_idx_and_gather(u, b):
            t, g = unit_coords(u)
            off = t * _B + g * _CHUNK
            pltpu.sync_copy(idx_hbm.at[pl.ds(off, _CHUNK)], idx_v[b])
            pltpu.async_copy(table_hbm.at[idx_v[b]], rows_v[b], gsem[b])

        def transpose_unit(b):
            def jj_body(jj, carry):
                row_base = jj * 128
                dst_base = jj * 1024
                for tr in range(4):
                    ob = outb[b].at[tr]
                    for s in range(8):
                        col = tr * 8 + s
                        for lblk in range(8):
                            src_rows = iota16 + (row_base + lblk * 16)
                            cols = jnp.full((16,), col, jnp.int32)
                            vec = plsc.load_gather(rows_v[b], [src_rows, cols])
                            ob[pl.ds(dst_base + s * 128 + lblk * 16, 16)] = vec
                return carry

            lax.fori_loop(0, 4, jj_body, 0)

        def start_writebacks(u, b):
            t, g = unit_coords(u)
            for tr in range(4):
                woff = t * 524288 + tr * 131072 + g * 4096
                pltpu.async_copy(outb[b].at[tr], out_hbm.at[pl.ds(woff, 4096)], wsem[b])

        def wait_writebacks(u, b):
            t, g = unit_coords(u)
            for tr in range(4):
                woff = t * 524288 + tr * 131072 + g * 4096
                pltpu.make_async_copy(
                    outb[b].at[tr], out_hbm.at[pl.ds(woff, 4096)], wsem[b]
                ).wait()

        def wait_gather(b):
            pltpu.make_async_copy(table_hbm.at[idx_v[b]], rows_v[b], gsem[b]).wait()

        for b in range(_NBUF):
            stage_idx_and_gather(u0 + b, b)

        def outer(gp, carry):
            for b in range(_NBUF):
                u = u0 + gp * _NBUF + b
                wait_gather(b)

                @pl.when(gp >= 1)
                def _():
                    wait_writebacks(u - _NBUF, b)

                transpose_unit(b)
                start_writebacks(u, b)
                stage_idx_and_gather(u + _NBUF, b)
            return carry

        n_main = per_w // _NBUF - 1
        lax.fori_loop(0, n_main, outer, 0)

        for b in range(_NBUF):
            u = u0 + per_w - _NBUF + b
            wait_gather(b)
            wait_writebacks(u - _NBUF, b)
            transpose_unit(b)
            start_writebacks(u, b)
        for b in range(_NBUF):
            u = u0 + per_w - _NBUF + b
            wait_writebacks(u, b)

    return gather_kernel(idx_flat, table)


def kernel(def_sens, embed_weight):
    idx_flat = def_sens.T.reshape(-1)
    flat = _gather_tiled(idx_flat, embed_weight)
    q5 = flat.reshape(_T, 4, _B // 128, 8, 128)
    return q5.transpose(2, 4, 0, 1, 3).reshape(_B, _T, _D)


# speedup vs baseline: 1.5542x; 1.5542x over previous
import functools

import jax
import jax.numpy as jnp
from jax import lax
from jax.experimental import pallas as pl
from jax.experimental.pallas import tpu as pltpu
from jax.experimental.pallas import tpu_sc as plsc

_NUM_WORKERS = 32
_CHUNK = 512
_NBUF = 2

_T = 200
_B = 16384
_D = 32


@functools.partial(jax.jit, static_argnums=())
def _gather_tiled(idx_flat, table):
    n_units = _T * (_B // _CHUNK)
    per_w = n_units // _NUM_WORKERS
    blocks_per_t = _B // _CHUNK
    n_out = _T * _D * _B
    mesh = plsc.VectorSubcoreMesh(core_axis_name="c", subcore_axis_name="s")

    scratch = (
        [pltpu.VMEM((_CHUNK,), jnp.int32) for _ in range(_NBUF)]
        + [pltpu.VMEM((_CHUNK, _D), jnp.float32) for _ in range(_NBUF)]
        + [pltpu.VMEM((16384,), jnp.float32) for _ in range(_NBUF)]
        + [pltpu.SemaphoreType.DMA for _ in range(2 * _NBUF)]
    )

    @functools.partial(
        pl.kernel,
        out_type=jax.ShapeDtypeStruct((n_out,), jnp.float32),
        mesh=mesh,
        scratch_types=scratch,
        compiler_params=pltpu.CompilerParams(
            use_tc_tiling_on_sc=False, needs_layout_passes=False
        ),
    )
    def gather_kernel(idx_hbm, table_hbm, out_hbm, *bufs):
        idx_v = bufs[:_NBUF]
        rows_v = bufs[_NBUF : 2 * _NBUF]
        outb = bufs[2 * _NBUF : 3 * _NBUF]
        gsem = bufs[3 * _NBUF : 4 * _NBUF]
        wsem = bufs[4 * _NBUF :]

        wid = lax.axis_index("s") * 2 + lax.axis_index("c")
        u0 = wid * per_w
        iota16 = lax.iota(jnp.int32, 16)

        def unit_coords(u):
            t = u // blocks_per_t
            g = u % blocks_per_t
            return t, g

        def stage_idx_and_gather(u, b):
            t, g = unit_coords(u)
            off = t * _B + g * _CHUNK
            pltpu.sync_copy(idx_hbm.at[pl.ds(off, _CHUNK)], idx_v[b])
            pltpu.async_copy(table_hbm.at[idx_v[b]], rows_v[b], gsem[b])

        col_splats = [jnp.full((16,), c, jnp.int32) for c in range(_D)]

        def transpose_unit(b):
            rows = rows_v[b]
            obf = outb[b]

            @plsc.parallel_loop(0, 32, 1)
            def _(m):
                jj = m >> 3
                lblk = m & 7
                row_base = jj * 128 + lblk * 16
                dst_base = jj * 1024 + lblk * 16
                rv = iota16 + row_base
                for tr in range(4):
                    for s in range(8):
                        vec = plsc.load_gather(rows, [rv, col_splats[tr * 8 + s]])
                        obf[pl.ds(dst_base + tr * 4096 + s * 128, 16)] = vec

        def start_writebacks(u, b):
            t, g = unit_coords(u)
            for tr in range(4):
                woff = t * 524288 + tr * 131072 + g * 4096
                pltpu.async_copy(
                    outb[b].at[pl.ds(tr * 4096, 4096)],
                    out_hbm.at[pl.ds(woff, 4096)],
                    wsem[b],
                )

        def wait_writebacks(u, b):
            t, g = unit_coords(u)
            for tr in range(4):
                woff = t * 524288 + tr * 131072 + g * 4096
                pltpu.make_async_copy(
                    outb[b].at[pl.ds(tr * 4096, 4096)],
                    out_hbm.at[pl.ds(woff, 4096)],
                    wsem[b],
                ).wait()

        def wait_gather(b):
            pltpu.make_async_copy(table_hbm.at[idx_v[b]], rows_v[b], gsem[b]).wait()

        for b in range(_NBUF):
            stage_idx_and_gather(u0 + b, b)

        def outer(gp, carry):
            for b in range(_NBUF):
                u = u0 + gp * _NBUF + b
                wait_gather(b)

                @pl.when(gp >= 1)
                def _():
                    wait_writebacks(u - _NBUF, b)

                transpose_unit(b)
                start_writebacks(u, b)
                stage_idx_and_gather(u + _NBUF, b)
            return carry

        n_main = per_w // _NBUF - 1
        lax.fori_loop(0, n_main, outer, 0)

        for b in range(_NBUF):
            u = u0 + per_w - _NBUF + b
            wait_gather(b)
            wait_writebacks(u - _NBUF, b)
            transpose_unit(b)
            start_writebacks(u, b)
        for b in range(_NBUF):
            u = u0 + per_w - _NBUF + b
            wait_writebacks(u, b)

    return gather_kernel(idx_flat, table)


def kernel(def_sens, embed_weight):
    idx_flat = def_sens.T.reshape(-1)
    flat = _gather_tiled(idx_flat, embed_weight)
    q5 = flat.reshape(_T, 4, _B // 128, 8, 128)
    return q5.transpose(2, 4, 0, 1, 3).reshape(_B, _T, _D)


# speedup vs baseline: 3.0377x; 1.9545x over previous
import functools

import jax
import jax.numpy as jnp
from jax import lax
from jax.experimental import pallas as pl
from jax.experimental.pallas import tpu as pltpu
from jax.experimental.pallas import tpu_sc as plsc

_NUM_WORKERS = 32
_CHUNK = 512
_NBUF = 2

_T = 200
_B = 16384
_D = 32

_SPITCH = 129
_TRROWS = 40


@jax.jit
def _gather_tiled(idx_flat, table):
    n_units = _T * (_B // _CHUNK)
    per_w = n_units // _NUM_WORKERS
    blocks_per_t = _B // _CHUNK
    n_out_rows = _T * _D * _B // 128
    mesh = plsc.VectorSubcoreMesh(core_axis_name="c", subcore_axis_name="s")

    scratch = (
        [pltpu.VMEM((_CHUNK,), jnp.int32) for _ in range(_NBUF)]
        + [pltpu.VMEM((_CHUNK, _D), jnp.float32) for _ in range(_NBUF)]
        + [pltpu.VMEM((4 * _TRROWS, _SPITCH), jnp.float32) for _ in range(_NBUF)]
        + [pltpu.SemaphoreType.DMA for _ in range(2 * _NBUF)]
    )

    @functools.partial(
        pl.kernel,
        out_type=jax.ShapeDtypeStruct((n_out_rows, 128), jnp.float32),
        mesh=mesh,
        scratch_types=scratch,
        compiler_params=pltpu.CompilerParams(
            use_tc_tiling_on_sc=False, needs_layout_passes=False
        ),
    )
    def gather_kernel(idx_hbm, table_hbm, out_hbm, *bufs):
        idx_v = bufs[:_NBUF]
        rows_v = bufs[_NBUF : 2 * _NBUF]
        outb = bufs[2 * _NBUF : 3 * _NBUF]
        gsem = bufs[3 * _NBUF : 4 * _NBUF]
        wsem = bufs[4 * _NBUF :]

        wid = lax.axis_index("s") * 2 + lax.axis_index("c")
        u0 = wid * per_w
        iota16 = lax.iota(jnp.int32, 16)
        r16 = (iota16 >> 3) * _TRROWS + (iota16 & 7)

        def unit_coords(u):
            t = u // blocks_per_t
            g = u % blocks_per_t
            return t, g

        def stage_idx_and_gather(u, b):
            t, g = unit_coords(u)
            off = t * _B + g * _CHUNK
            pltpu.sync_copy(idx_hbm.at[pl.ds(off, _CHUNK)], idx_v[b])
            pltpu.async_copy(table_hbm.at[idx_v[b]], rows_v[b], gsem[b])

        def wait_gather(b):
            pltpu.make_async_copy(table_hbm.at[idx_v[b]], rows_v[b], gsem[b]).wait()

        def transpose_unit(b):
            rows = rows_v[b]
            ob = outb[b]

            @plsc.parallel_loop(0, _CHUNK, 1, unroll=2)
            def _(j):
                jj = j >> 7
                l = j & 127
                cvec = jnp.full((16,), 0, jnp.int32) + l
                base = jj * 8
                for half in range(2):
                    vec = rows[j, pl.ds(half * 16, 16)]
                    rvec = r16 + (base + half * 2 * _TRROWS)
                    plsc.store_scatter(ob, [rvec, cvec], vec)

        def writebacks(u, b, wait):
            t, g = unit_coords(u)
            for tr in range(4):
                for jj in range(4):
                    src = outb[b].at[pl.ds(tr * _TRROWS + jj * 8, 8), pl.ds(0, 128)]
                    row0 = t * 4096 + tr * 1024 + (g * 4 + jj) * 8
                    dst = out_hbm.at[pl.ds(row0, 8), pl.ds(0, 128)]
                    if wait:
                        pltpu.make_async_copy(src, dst, wsem[b]).wait()
                    else:
                        pltpu.async_copy(src, dst, wsem[b])

        for b in range(_NBUF):
            stage_idx_and_gather(u0 + b, b)

        def outer(gp, carry):
            for b in range(_NBUF):
                u = u0 + gp * _NBUF + b
                wait_gather(b)

                @pl.when(gp >= 1)
                def _():
                    writebacks(u - _NBUF, b, wait=True)

                transpose_unit(b)
                writebacks(u, b, wait=False)
                stage_idx_and_gather(u + _NBUF, b)
            return carry

        n_main = per_w // _NBUF - 1
        lax.fori_loop(0, n_main, outer, 0)

        for b in range(_NBUF):
            u = u0 + per_w - _NBUF + b
            wait_gather(b)
            writebacks(u - _NBUF, b, wait=True)
            transpose_unit(b)
            writebacks(u, b, wait=False)
        for b in range(_NBUF):
            u = u0 + per_w - _NBUF + b
            writebacks(u, b, wait=True)

    return gather_kernel(idx_flat, table)


def kernel(def_sens, embed_weight):
    idx_flat = def_sens.T.reshape(-1)
    flat = _gather_tiled(idx_flat, embed_weight)
    q5 = flat.reshape(_T, 4, _B // 128, 8, 128)
    return q5.transpose(2, 4, 0, 1, 3).reshape(_B, _T, _D)
